# Initial kernel scaffold; baseline (speedup 1.0000x reference)
import functools
import jax
import jax.numpy as jnp
from jax import lax
from jax.experimental import pallas as pl
from jax.experimental.pallas import tpu as pltpu
from jax.experimental.pallas import tpu_sc as plsc

N = 10000
D = 128
E = 320000
NC = 2
NS = 16
NW = NC * NS
CHUNK = 125
EDGES_PER_TILE = E // NW
CH = EDGES_PER_TILE // CHUNK
RPT = N // NS

_mesh = plsc.VectorSubcoreMesh(
    core_axis_name="c", subcore_axis_name="s", num_cores=NC, num_subcores=NS
)


def _deg_body(dst_hbm, ones_hbm, zeros_hbm, out_hbm, deg_sh, dst_v, ones_v):
    c = lax.axis_index("c")
    s = lax.axis_index("s")
    wid = c * NS + s
    pltpu.sync_copy(dst_hbm.at[pl.ds(wid * CH, CH)], dst_v)
    pltpu.sync_copy(ones_hbm, ones_v)
    pltpu.sync_copy(
        zeros_hbm.at[pl.ds(s * RPT, RPT)], deg_sh.at[pl.ds(s * RPT, RPT)]
    )
    plsc.subcore_barrier()

    def body(j, carry):
        pltpu.sync_copy(ones_v, deg_sh.at[dst_v.at[j]], add=True)
        return carry

    lax.fori_loop(0, CH, body, 0)
    plsc.subcore_barrier()
    pltpu.sync_copy(
        deg_sh.at[pl.ds(s * RPT, RPT)], out_hbm.at[c].at[pl.ds(s * RPT, RPT)]
    )


_deg_call = functools.partial(
    pl.kernel,
    out_type=jax.ShapeDtypeStruct((NC, N, 16), jnp.float32),
    mesh=_mesh,
    scratch_types=[
        pltpu.VMEM_SHARED((N, 16), jnp.float32),
        pltpu.VMEM((CH, CHUNK), jnp.int32),
        pltpu.VMEM((CHUNK, 16), jnp.float32),
    ],
)(_deg_body)


def _msg_body(y_hbm, src_hbm, dst_hbm, zeros_hbm, out_hbm,
              acc_sh, src_v, dst_v, rows_v, sem):
    c = lax.axis_index("c")
    s = lax.axis_index("s")
    wid = c * NS + s
    pltpu.sync_copy(src_hbm.at[pl.ds(wid * CH, CH)], src_v)
    pltpu.sync_copy(dst_hbm.at[pl.ds(wid * CH, CH)], dst_v)
    pltpu.sync_copy(
        zeros_hbm.at[pl.ds(s * RPT, RPT)], acc_sh.at[pl.ds(s * RPT, RPT)]
    )
    plsc.subcore_barrier()

    def body(j, carry):
        pltpu.async_copy(y_hbm.at[src_v.at[j]], rows_v, sem).wait()
        pltpu.sync_copy(rows_v, acc_sh.at[dst_v.at[j]], add=True)
        return carry

    lax.fori_loop(0, CH, body, 0)
    plsc.subcore_barrier()
    pltpu.sync_copy(
        acc_sh.at[pl.ds(s * RPT, RPT)], out_hbm.at[c].at[pl.ds(s * RPT, RPT)]
    )


_msg_call = functools.partial(
    pl.kernel,
    out_type=jax.ShapeDtypeStruct((NC, N, D), jnp.float32),
    mesh=_mesh,
    scratch_types=[
        pltpu.VMEM_SHARED((N, D), jnp.float32),
        pltpu.VMEM((CH, CHUNK), jnp.int32),
        pltpu.VMEM((CH, CHUNK), jnp.int32),
        pltpu.VMEM((CHUNK, D), jnp.float32),
        pltpu.SemaphoreType.DMA,
    ],
)(_msg_body)


R = 1000


def _dinv(degp_ref):
    deg = degp_ref[0] + degp_ref[1] + 1.0
    return lax.rsqrt(deg)[:, 0:1]


def _lin_body(x_ref, w_ref, degp_ref, y_ref):
    y_ref[...] = (
        jnp.dot(x_ref[...], w_ref[...], preferred_element_type=jnp.float32)
        * _dinv(degp_ref)
    )


def _mid_body(acc_ref, y_ref, degp_ref, b_ref, w_ref, out_ref):
    dinv = _dinv(degp_ref)
    h = jnp.maximum(
        dinv * (acc_ref[0] + acc_ref[1] + y_ref[...]) + b_ref[...], 0.0
    )
    out_ref[...] = (
        jnp.dot(h, w_ref[...], preferred_element_type=jnp.float32) * dinv
    )


def _fin_body(acc_ref, y_ref, degp_ref, b_ref, out_ref):
    dinv = _dinv(degp_ref)
    out_ref[...] = dinv * (acc_ref[0] + acc_ref[1] + y_ref[...]) + b_ref[...]


_row_spec = pl.BlockSpec((R, D), lambda i: (i, 0))
_acc_spec = pl.BlockSpec((NC, R, D), lambda i: (0, i, 0))
_deg_spec = pl.BlockSpec((NC, R, 16), lambda i: (0, i, 0))
_w_spec = pl.BlockSpec((D, D), lambda i: (0, 0))
_b_spec = pl.BlockSpec((1, D), lambda i: (0, 0))

_lin_call = pl.pallas_call(
    _lin_body,
    grid=(N // R,),
    in_specs=[_row_spec, _w_spec, _deg_spec],
    out_specs=_row_spec,
    out_shape=jax.ShapeDtypeStruct((N, D), jnp.float32),
)

_mid_call = pl.pallas_call(
    _mid_body,
    grid=(N // R,),
    in_specs=[_acc_spec, _row_spec, _deg_spec, _b_spec, _w_spec],
    out_specs=_row_spec,
    out_shape=jax.ShapeDtypeStruct((N, D), jnp.float32),
)

_fin_call = pl.pallas_call(
    _fin_body,
    grid=(N // R,),
    in_specs=[_acc_spec, _row_spec, _deg_spec, _b_spec],
    out_specs=_row_spec,
    out_shape=jax.ShapeDtypeStruct((N, D), jnp.float32),
)


@jax.jit
def kernel(x, edge_index, W1, b1, Wh, bh, W2, b2):
    ei = edge_index.astype(jnp.int32)
    src = ei[0].reshape(NW * CH, CHUNK)
    dst = ei[1].reshape(NW * CH, CHUNK)
    zeros_d = jnp.zeros((N, D), jnp.float32)
    zeros_16 = jnp.zeros((N, 16), jnp.float32)
    ones_16 = jnp.ones((CHUNK, 16), jnp.float32)

    degp = _deg_call(dst, ones_16, zeros_16)
    y1 = _lin_call(x, W1, degp)
    acc1 = _msg_call(y1, src, dst, zeros_d)
    y2 = _mid_call(acc1, y1, degp, b1.reshape(1, D), Wh)
    acc2 = _msg_call(y2, src, dst, zeros_d)
    y3 = _mid_call(acc2, y2, degp, bh.reshape(1, D), W2)
    acc3 = _msg_call(y3, src, dst, zeros_d)
    out = _fin_call(acc3, y3, degp, b2.reshape(1, D))
    return out


# speedup vs baseline: 7.1917x; 7.1917x over previous
import functools
import jax
import jax.numpy as jnp
from jax import lax
from jax.experimental import pallas as pl
from jax.experimental.pallas import tpu as pltpu
from jax.experimental.pallas import tpu_sc as plsc

N = 10000
D = 128
E = 320000
NC = 2
NS = 16
NW = NC * NS
CHUNK = 128
CH = 80
EP = NW * CH * CHUNK
NP = 10240
PAD_DST = N
RPT = NP // NS

_mesh = plsc.VectorSubcoreMesh(
    core_axis_name="c", subcore_axis_name="s", num_cores=NC, num_subcores=NS
)


def _deg_body(dst_hbm, ones_hbm, zeros_hbm, out_hbm, deg_sh, dst_v, ones_v):
    c = lax.axis_index("c")
    s = lax.axis_index("s")
    wid = c * NS + s
    pltpu.sync_copy(dst_hbm.at[pl.ds(wid * CH, CH)], dst_v)
    pltpu.sync_copy(ones_hbm, ones_v)
    pltpu.sync_copy(
        zeros_hbm.at[pl.ds(s * RPT, RPT)], deg_sh.at[pl.ds(s * RPT, RPT)]
    )
    plsc.subcore_barrier()

    def body(j, carry):
        pltpu.sync_copy(ones_v, deg_sh.at[dst_v.at[j]], add=True)
        return carry

    lax.fori_loop(0, CH, body, 0)
    plsc.subcore_barrier()
    pltpu.sync_copy(
        deg_sh.at[pl.ds(s * RPT, RPT)], out_hbm.at[c].at[pl.ds(s * RPT, RPT)]
    )


_deg_call = functools.partial(
    pl.kernel,
    out_type=jax.ShapeDtypeStruct((NC, NP, D), jnp.float32),
    mesh=_mesh,
    scratch_types=[
        pltpu.VMEM_SHARED((NP, D), jnp.float32),
        pltpu.VMEM((CH, CHUNK), jnp.int32),
        pltpu.VMEM((CHUNK, D), jnp.float32),
    ],
)(_deg_body)


def _msg_body(y_hbm, src_hbm, dst_hbm, zeros_hbm, out_hbm,
              acc_sh, src_v, dst_v, rows_v, sem):
    c = lax.axis_index("c")
    s = lax.axis_index("s")
    wid = c * NS + s
    pltpu.sync_copy(src_hbm.at[pl.ds(wid * CH, CH)], src_v)
    pltpu.sync_copy(dst_hbm.at[pl.ds(wid * CH, CH)], dst_v)
    pltpu.sync_copy(
        zeros_hbm.at[pl.ds(s * RPT, RPT)], acc_sh.at[pl.ds(s * RPT, RPT)]
    )
    plsc.subcore_barrier()

    def body(j, carry):
        pltpu.async_copy(y_hbm.at[src_v.at[j]], rows_v, sem).wait()
        pltpu.sync_copy(rows_v, acc_sh.at[dst_v.at[j]], add=True)
        return carry

    lax.fori_loop(0, CH, body, 0)
    plsc.subcore_barrier()
    pltpu.sync_copy(
        acc_sh.at[pl.ds(s * RPT, RPT)], out_hbm.at[c].at[pl.ds(s * RPT, RPT)]
    )


_msg_call = functools.partial(
    pl.kernel,
    out_type=jax.ShapeDtypeStruct((NC, NP, D), jnp.float32),
    mesh=_mesh,
    scratch_types=[
        pltpu.VMEM_SHARED((NP, D), jnp.float32),
        pltpu.VMEM((CH, CHUNK), jnp.int32),
        pltpu.VMEM((CH, CHUNK), jnp.int32),
        pltpu.VMEM((CHUNK, D), jnp.float32),
        pltpu.SemaphoreType.DMA,
    ],
)(_msg_body)


R = 1000


def _dinv(degp_ref):
    deg = degp_ref[0, :, 0:1] + degp_ref[1, :, 0:1] + 1.0
    return lax.rsqrt(deg)


def _lin_body(x_ref, w_ref, degp_ref, y_ref):
    y_ref[...] = (
        jnp.dot(x_ref[...], w_ref[...], preferred_element_type=jnp.float32)
        * _dinv(degp_ref)
    )


def _mid_body(acc_ref, y_ref, degp_ref, b_ref, w_ref, out_ref):
    dinv = _dinv(degp_ref)
    h = jnp.maximum(
        dinv * (acc_ref[0] + acc_ref[1] + y_ref[...]) + b_ref[...], 0.0
    )
    out_ref[...] = (
        jnp.dot(h, w_ref[...], preferred_element_type=jnp.float32) * dinv
    )


def _fin_body(acc_ref, y_ref, degp_ref, b_ref, out_ref):
    dinv = _dinv(degp_ref)
    out_ref[...] = dinv * (acc_ref[0] + acc_ref[1] + y_ref[...]) + b_ref[...]


_row_spec = pl.BlockSpec((R, D), lambda i: (i, 0))
_acc_spec = pl.BlockSpec((NC, R, D), lambda i: (0, i, 0))
_deg_spec = pl.BlockSpec((NC, R, D), lambda i: (0, i, 0))
_w_spec = pl.BlockSpec((D, D), lambda i: (0, 0))
_b_spec = pl.BlockSpec((1, D), lambda i: (0, 0))

_lin_call = pl.pallas_call(
    _lin_body,
    grid=(N // R,),
    in_specs=[_row_spec, _w_spec, _deg_spec],
    out_specs=_row_spec,
    out_shape=jax.ShapeDtypeStruct((N, D), jnp.float32),
)

_mid_call = pl.pallas_call(
    _mid_body,
    grid=(N // R,),
    in_specs=[_acc_spec, _row_spec, _deg_spec, _b_spec, _w_spec],
    out_specs=_row_spec,
    out_shape=jax.ShapeDtypeStruct((N, D), jnp.float32),
)

_fin_call = pl.pallas_call(
    _fin_body,
    grid=(N // R,),
    in_specs=[_acc_spec, _row_spec, _deg_spec, _b_spec],
    out_specs=_row_spec,
    out_shape=jax.ShapeDtypeStruct((N, D), jnp.float32),
)


@jax.jit
def kernel(x, edge_index, W1, b1, Wh, bh, W2, b2):
    ei = edge_index.astype(jnp.int32)
    src = jnp.concatenate(
        [ei[0], jnp.zeros((EP - E,), jnp.int32)]
    ).reshape(NW * CH, CHUNK)
    dst = jnp.concatenate(
        [ei[1], jnp.full((EP - E,), PAD_DST, jnp.int32)]
    ).reshape(NW * CH, CHUNK)
    zeros_d = jnp.zeros((NP, D), jnp.float32)
    ones_d = jnp.ones((CHUNK, D), jnp.float32)

    degp = _deg_call(dst, ones_d, zeros_d)
    y1 = _lin_call(x, W1, degp)
    acc1 = _msg_call(y1, src, dst, zeros_d)
    y2 = _mid_call(acc1, y1, degp, b1.reshape(1, D), Wh)
    acc2 = _msg_call(y2, src, dst, zeros_d)
    y3 = _mid_call(acc2, y2, degp, bh.reshape(1, D), W2)
    acc3 = _msg_call(y3, src, dst, zeros_d)
    out = _fin_call(acc3, y3, degp, b2.reshape(1, D))
    return out


# speedup vs baseline: 7.6100x; 1.0582x over previous
import functools
import jax
import jax.numpy as jnp
from jax import lax
from jax.experimental import pallas as pl
from jax.experimental.pallas import tpu as pltpu
from jax.experimental.pallas import tpu_sc as plsc

N = 10000
D = 128
E = 320000
NC = 2
NS = 16
NW = NC * NS
CHUNK = 128
CH = 80
EP = NW * CH * CHUNK
NP = 10112
PAD_DST = N
RPT = NP // NS

_mesh = plsc.VectorSubcoreMesh(
    core_axis_name="c", subcore_axis_name="s", num_cores=NC, num_subcores=NS
)


def _deg_body(dst_hbm, ones_hbm, zeros_hbm, out_hbm, deg_sh, dst_v, ones_v):
    c = lax.axis_index("c")
    s = lax.axis_index("s")
    wid = c * NS + s
    pltpu.sync_copy(dst_hbm.at[pl.ds(wid * CH, CH)], dst_v)
    pltpu.sync_copy(ones_hbm, ones_v)
    pltpu.sync_copy(
        zeros_hbm.at[pl.ds(s * RPT, RPT)], deg_sh.at[pl.ds(s * RPT, RPT)]
    )
    plsc.subcore_barrier()

    def body(j, carry):
        pltpu.sync_copy(ones_v, deg_sh.at[dst_v.at[j]], add=True)
        return carry

    lax.fori_loop(0, CH, body, 0)
    plsc.subcore_barrier()
    pltpu.sync_copy(
        deg_sh.at[pl.ds(s * RPT, RPT)], out_hbm.at[c].at[pl.ds(s * RPT, RPT)]
    )


_deg_call = functools.partial(
    pl.kernel,
    out_type=jax.ShapeDtypeStruct((NC, NP, D), jnp.float32),
    mesh=_mesh,
    scratch_types=[
        pltpu.VMEM_SHARED((NP, D), jnp.float32),
        pltpu.VMEM((CH, CHUNK), jnp.int32),
        pltpu.VMEM((CHUNK, D), jnp.float32),
    ],
)(_deg_body)


def _msg_body(y_hbm, src_hbm, dst_hbm, zeros_hbm, out_hbm,
              acc_sh, src_v, dst_v, rows0_v, rows1_v, sem0, sem1):
    c = lax.axis_index("c")
    s = lax.axis_index("s")
    wid = c * NS + s
    pltpu.sync_copy(
        zeros_hbm.at[pl.ds(s * RPT, RPT)], acc_sh.at[pl.ds(s * RPT, RPT)]
    )
    plsc.subcore_barrier()

    HC = CH // 2
    for h in range(2):
        pltpu.sync_copy(src_hbm.at[pl.ds(wid * CH + h * HC, HC)], src_v)
        pltpu.sync_copy(dst_hbm.at[pl.ds(wid * CH + h * HC, HC)], dst_v)
        pltpu.async_copy(y_hbm.at[src_v.at[0]], rows0_v, sem0)
        pltpu.async_copy(y_hbm.at[src_v.at[1]], rows1_v, sem1)

        def body(i, carry):
            j0 = 2 * i
            j1 = j0 + 1
            pltpu.make_async_copy(y_hbm.at[src_v.at[j0]], rows0_v, sem0).wait()
            pltpu.sync_copy(rows0_v, acc_sh.at[dst_v.at[j0]], add=True)

            @pl.when(i < HC // 2 - 1)
            def _():
                pltpu.async_copy(y_hbm.at[src_v.at[j0 + 2]], rows0_v, sem0)

            pltpu.make_async_copy(y_hbm.at[src_v.at[j1]], rows1_v, sem1).wait()
            pltpu.sync_copy(rows1_v, acc_sh.at[dst_v.at[j1]], add=True)

            @pl.when(i < HC // 2 - 1)
            def _():
                pltpu.async_copy(y_hbm.at[src_v.at[j1 + 2]], rows1_v, sem1)

            return carry

        lax.fori_loop(0, HC // 2, body, 0)
    plsc.subcore_barrier()
    pltpu.sync_copy(
        acc_sh.at[pl.ds(s * RPT, RPT)], out_hbm.at[c].at[pl.ds(s * RPT, RPT)]
    )


_msg_call = functools.partial(
    pl.kernel,
    out_type=jax.ShapeDtypeStruct((NC, NP, D), jnp.float32),
    mesh=_mesh,
    scratch_types=[
        pltpu.VMEM_SHARED((NP, D), jnp.float32),
        pltpu.VMEM((CH // 2, CHUNK), jnp.int32),
        pltpu.VMEM((CH // 2, CHUNK), jnp.int32),
        pltpu.VMEM((CHUNK, D), jnp.float32),
        pltpu.VMEM((CHUNK, D), jnp.float32),
        pltpu.SemaphoreType.DMA,
        pltpu.SemaphoreType.DMA,
    ],
)(_msg_body)


R = 1000


def _dinv(degp_ref):
    deg = degp_ref[0, :, 0:1] + degp_ref[1, :, 0:1] + 1.0
    return lax.rsqrt(deg)


def _lin_body(x_ref, w_ref, degp_ref, y_ref):
    y_ref[...] = (
        jnp.dot(x_ref[...], w_ref[...], preferred_element_type=jnp.float32)
        * _dinv(degp_ref)
    )


def _mid_body(acc_ref, y_ref, degp_ref, b_ref, w_ref, out_ref):
    dinv = _dinv(degp_ref)
    h = jnp.maximum(
        dinv * (acc_ref[0] + acc_ref[1] + y_ref[...]) + b_ref[...], 0.0
    )
    out_ref[...] = (
        jnp.dot(h, w_ref[...], preferred_element_type=jnp.float32) * dinv
    )


def _fin_body(acc_ref, y_ref, degp_ref, b_ref, out_ref):
    dinv = _dinv(degp_ref)
    out_ref[...] = dinv * (acc_ref[0] + acc_ref[1] + y_ref[...]) + b_ref[...]


_row_spec = pl.BlockSpec((R, D), lambda i: (i, 0))
_acc_spec = pl.BlockSpec((NC, R, D), lambda i: (0, i, 0))
_deg_spec = pl.BlockSpec((NC, R, D), lambda i: (0, i, 0))
_w_spec = pl.BlockSpec((D, D), lambda i: (0, 0))
_b_spec = pl.BlockSpec((1, D), lambda i: (0, 0))

_lin_call = pl.pallas_call(
    _lin_body,
    grid=(N // R,),
    in_specs=[_row_spec, _w_spec, _deg_spec],
    out_specs=_row_spec,
    out_shape=jax.ShapeDtypeStruct((N, D), jnp.float32),
)

_mid_call = pl.pallas_call(
    _mid_body,
    grid=(N // R,),
    in_specs=[_acc_spec, _row_spec, _deg_spec, _b_spec, _w_spec],
    out_specs=_row_spec,
    out_shape=jax.ShapeDtypeStruct((N, D), jnp.float32),
)

_fin_call = pl.pallas_call(
    _fin_body,
    grid=(N // R,),
    in_specs=[_acc_spec, _row_spec, _deg_spec, _b_spec],
    out_specs=_row_spec,
    out_shape=jax.ShapeDtypeStruct((N, D), jnp.float32),
)


@jax.jit
def kernel(x, edge_index, W1, b1, Wh, bh, W2, b2):
    ei = edge_index.astype(jnp.int32)
    src = jnp.concatenate(
        [ei[0], jnp.zeros((EP - E,), jnp.int32)]
    ).reshape(NW * CH, CHUNK)
    dst = jnp.concatenate(
        [ei[1], jnp.full((EP - E,), PAD_DST, jnp.int32)]
    ).reshape(NW * CH, CHUNK)
    zeros_d = jnp.zeros((NP, D), jnp.float32)
    ones_d = jnp.ones((CHUNK, D), jnp.float32)

    degp = _deg_call(dst, ones_d, zeros_d)
    y1 = _lin_call(x, W1, degp)
    acc1 = _msg_call(y1, src, dst, zeros_d)
    y2 = _mid_call(acc1, y1, degp, b1.reshape(1, D), Wh)
    acc2 = _msg_call(y2, src, dst, zeros_d)
    y3 = _mid_call(acc2, y2, degp, bh.reshape(1, D), W2)
    acc3 = _msg_call(y3, src, dst, zeros_d)
    out = _fin_call(acc3, y3, degp, b2.reshape(1, D))
    return out
